# Initial kernel scaffold; baseline (speedup 1.0000x reference)
import jax
import jax.numpy as jnp
from jax.experimental import pallas as pl


def kernel(x, W_enc, W_dec, b_enc, b_dec):
    raise NotImplementedError("write your pallas kernel here")


# speedup vs baseline: 10.3938x; 10.3938x over previous
import functools

import jax
import jax.numpy as jnp
from jax import lax
from jax.experimental import pallas as pl
from jax.experimental.pallas import tpu as pltpu
from jax.experimental.pallas import tpu_sc as plsc

BATCH = 512
INST = 2
D_IN = 256
D_SAE = 16384
D_LVL0 = 4096
K_LVL0 = BATCH * INST * 32
K_LVL1 = BATCH * INST * 64

ROWS = BATCH * INST
N_TILES = 16
ROWS_PER_TILE = ROWS // N_TILES
LANES = 16
HIST = 65536
CHUNK = HIST // N_TILES
CHUNK_V = CHUNK // LANES


_BB = 256
_FB = 2048


def _enc_body(x_ref, w_ref, b_ref, out_ref):
    for i in range(INST):
        acc = lax.dot_general(
            x_ref[:, i, :], w_ref[i], (((1,), (1,)), ((), ())),
            preferred_element_type=jnp.float32)
        acc = acc + b_ref[i][None, :]
        out_ref[:, i, :] = jnp.maximum(acc, 0.0)


def _encoder(xf, w_enc, b_enc):
    nb = BATCH // _BB
    nf = D_SAE // _FB
    return pl.pallas_call(
        _enc_body,
        grid=(nb, nf),
        in_specs=[
            pl.BlockSpec((_BB, INST, D_IN), lambda b, f: (b, 0, 0)),
            pl.BlockSpec((INST, _FB, D_IN), lambda b, f: (0, f, 0)),
            pl.BlockSpec((INST, _FB), lambda b, f: (0, f)),
        ],
        out_specs=pl.BlockSpec((_BB, INST, _FB), lambda b, f: (b, 0, f)),
        out_shape=jax.ShapeDtypeStruct((BATCH, INST, D_SAE), jnp.float32),
    )(xf, w_enc, b_enc)


def _sc_select(acts2):
    mesh = plsc.VectorSubcoreMesh(core_axis_name="c", subcore_axis_name="s")
    kern = pl.kernel(
        _sc_select_wrapped,
        out_type=(
            jax.ShapeDtypeStruct((INST, LANES), jnp.float32),
            jax.ShapeDtypeStruct((INST, N_TILES, N_TILES + 1, CHUNK), jnp.int32),
        ),
        mesh=mesh,
        compiler_params=pltpu.CompilerParams(needs_layout_passes=False),
        scratch_types=[
            pltpu.VMEM((D_SAE,), jnp.float32),
            pltpu.VMEM((N_TILES, CHUNK), jnp.int32),
            pltpu.VMEM((CHUNK,), jnp.int32),
            pltpu.VMEM((CHUNK,), jnp.int32),
            pltpu.VMEM((LANES,), jnp.int32),
            pltpu.VMEM((N_TILES, LANES), jnp.int32),
            pltpu.VMEM((LANES,), jnp.int32),
            pltpu.VMEM((LANES,), jnp.float32),
        ],
    )
    return kern(acts2)[0]


def _sc_select_wrapped(acts_hbm, out_hbm, hist_hbm, buf, hist, merged, tmp,
                       totv, totv2, resv, thrv):
    core = lax.axis_index("c")
    sub = lax.axis_index("s")

    n_vregs_row = jnp.where(core == 0, D_LVL0 // LANES, D_SAE // LANES)
    k_target = jnp.where(core == 0, jnp.int32(K_LVL0), jnp.int32(K_LVL1))
    zeros16 = jnp.zeros((LANES,), jnp.int32)
    iota = lax.iota(jnp.int32, LANES)

    def zero_ref(ref, n_v):
        def zbody(j, _):
            ref[pl.ds(j * LANES, LANES)] = zeros16
            return 0
        lax.fori_loop(0, n_v, zbody, 0)

    def extract_lane(vec, lane):
        sel = jnp.where(iota == lane, vec, jnp.zeros_like(vec))
        return lax.reduce_sum_p.bind(sel, axes=(0,))

    def splat(x):
        return jnp.full((LANES,), x, dtype=jnp.int32)

    def zero_hist():
        for r in range(N_TILES):
            def zbody(j, _):
                hist[r, pl.ds(j * LANES, LANES)] = zeros16
                return 0
            lax.fori_loop(0, CHUNK_V, zbody, 0)

    def stream_histogram(mode, b1):
        zero_hist()
        b1v = splat(b1)

        def row_body(rr, _):
            r = sub * ROWS_PER_TILE + rr
            pltpu.sync_copy(acts_hbm.at[r], buf)

            def vbody(j, _):
                v = buf[pl.ds(j * LANES, LANES)]
                pos = v > 0.0
                bits = plsc.bitcast(v, jnp.int32)
                hi = lax.shift_right_logical(bits, 16)
                if mode == 0:
                    bkt = hi
                    m = pos
                else:
                    bkt = lax.bitwise_and(bits, jnp.int32(0xFFFF))
                    m = jnp.logical_and(pos, hi == b1v)
                cnt, last = plsc.scan_count(bkt, mask=m)
                row = lax.shift_right_logical(bkt, 12)
                col = lax.bitwise_and(bkt, jnp.int32(CHUNK - 1))
                plsc.addupdate_scatter(hist, [row, col], cnt, mask=last)
                return 0

            lax.fori_loop(0, n_vregs_row, vbody, 0)
            return 0

        lax.fori_loop(0, ROWS_PER_TILE, row_body, 0)

    def merge_and_scan(k_want):
        pltpu.sync_copy(hist, hist_hbm.at[core, sub, pl.ds(0, N_TILES)])
        plsc.subcore_barrier()

        zero_ref(merged, CHUNK_V)

        def mbody(t, _):
            pltpu.sync_copy(hist_hbm.at[core, t, sub], tmp)

            def abody(j, _):
                s = pl.ds(j * LANES, LANES)
                merged[s] = merged[s] + tmp[s]
                return 0

            lax.fori_loop(0, CHUNK_V, abody, 0)
            return 0

        lax.fori_loop(0, N_TILES, mbody, 0)

        def tbody(j, acc):
            return acc + merged[pl.ds(j * LANES, LANES)]

        tot_vec = lax.fori_loop(0, CHUNK_V, tbody, zeros16)
        chunk_tot = lax.reduce_sum_p.bind(tot_vec, axes=(0,))
        totv[pl.ds(0, LANES)] = splat(chunk_tot)
        pltpu.sync_copy(totv, hist_hbm.at[core, sub, N_TILES, pl.ds(0, LANES)])
        plsc.subcore_barrier()

        for t in range(N_TILES):
            pltpu.sync_copy(hist_hbm.at[core, t, N_TILES, pl.ds(0, LANES)],
                            totv2.at[t])

        def sbody(i, carry):
            acc, cstar, above = carry
            t = N_TILES - 1 - i
            tvec = totv2[t]
            tt = extract_lane(tvec, jnp.int32(0))
            new_acc = acc + tt
            found = jnp.logical_and(cstar < 0, new_acc >= k_want)
            cstar = jnp.where(found, t, cstar)
            above = jnp.where(found, acc, above)
            return new_acc, cstar, above

        _, cstar, above = lax.fori_loop(
            0, N_TILES, sbody,
            (jnp.int32(0), jnp.int32(-1), jnp.int32(0)))

        def cbody(i, carry):
            cum, bsel, cgt = carry
            jv = CHUNK_V - 1 - i
            v = merged[pl.ds(jv * LANES, LANES)]
            rv = lax.rev(v, (0,))
            cs = plsc.cumsum(rv)
            cumincl = splat(cum) + cs
            crossed = cumincl >= splat(k_want)
            any_crossed = lax.reduce_max_p.bind(
                crossed.astype(jnp.int32), axes=(0,)) > 0
            lane = plsc.all_reduce_ffs(crossed)
            if lane.ndim != 0:
                lane = extract_lane(lane, jnp.int32(0))
            found = jnp.logical_and(bsel < 0, any_crossed)
            bucket = jv * LANES + (LANES - 1 - lane)
            cnt_at = extract_lane(rv, lane)
            cumincl_at = extract_lane(cumincl, lane)
            bsel = jnp.where(found, bucket, bsel)
            cgt = jnp.where(found, cumincl_at - cnt_at, cgt)
            cum = cum + lax.reduce_sum_p.bind(rv, axes=(0,))
            return cum, bsel, cgt

        _, bsel, cgt = lax.fori_loop(
            0, CHUNK_V, cbody,
            (above, jnp.int32(-1), jnp.int32(0)))

        @pl.when(sub == 0)
        def _():
            resv[pl.ds(0, LANES)] = zeros16
            pltpu.sync_copy(resv, hist_hbm.at[core, 0, N_TILES, pl.ds(LANES, LANES)])

        plsc.subcore_barrier()

        @pl.when(jnp.logical_and(cstar >= 0, sub == cstar))
        def _():
            bucket_g = cstar * CHUNK + bsel
            vec = jnp.where(iota == 0, splat(bucket_g),
                            jnp.where(iota == 1, splat(cgt),
                                      jnp.where(iota == 2, splat(1),
                                                zeros16)))
            resv[pl.ds(0, LANES)] = vec
            pltpu.sync_copy(resv, hist_hbm.at[core, 0, N_TILES, pl.ds(LANES, LANES)])

        plsc.subcore_barrier()
        pltpu.sync_copy(hist_hbm.at[core, 0, N_TILES, pl.ds(LANES, LANES)], resv)
        vec = resv[pl.ds(0, LANES)]
        bucket = extract_lane(vec, jnp.int32(0))
        count_gt = extract_lane(vec, jnp.int32(1))
        valid = extract_lane(vec, jnp.int32(2))
        return bucket, count_gt, valid, cstar

    stream_histogram(0, jnp.int32(0))
    b1, cgt1, valid1, _ = merge_and_scan(k_target)

    k_rem = k_target - cgt1
    stream_histogram(1, b1)
    b2, _, valid2, cstar2 = merge_and_scan(k_rem)

    thr_bits = lax.bitwise_or(lax.shift_left(b1, 16), b2)
    thr_vec = plsc.bitcast(splat(thr_bits), jnp.float32)
    ok = jnp.logical_and(valid1 > 0, valid2 > 0)
    thr_vec = jnp.where(ok, thr_vec, jnp.zeros((LANES,), jnp.float32))
    writer = jnp.where(cstar2 >= 0, cstar2, jnp.int32(0))

    @pl.when(sub == writer)
    def _():
        thrv[pl.ds(0, LANES)] = thr_vec
        pltpu.sync_copy(thrv, out_hbm.at[core])


def _dec_body(nf_total, acts_ref, w_ref, b_ref, thr_ref, topk_ref, recon_ref):
    nf = pl.program_id(1)
    t = thr_ref[0, 0]
    for i in range(INST):
        a = acts_ref[:, i, :]
        masked = jnp.where(a >= t, a, 0.0)
        topk_ref[:, i, :] = masked
        part = lax.dot_general(
            masked, w_ref[i], (((1,), (1,)), ((), ())),
            preferred_element_type=jnp.float32)

        @pl.when(nf == 0)
        def _():
            recon_ref[:, i, :] = part

        @pl.when(nf > 0)
        def _():
            recon_ref[:, i, :] += part

        @pl.when(nf == nf_total - 1)
        def _():
            recon_ref[:, i, :] = jnp.maximum(
                recon_ref[:, i, :] + b_ref[i][None, :], 0.0)


def _decoder(acts, w_dec, b_dec, thr, d_level):
    nb = BATCH // _BB
    nf = d_level // _FB
    return pl.pallas_call(
        functools.partial(_dec_body, nf),
        grid=(nb, nf),
        in_specs=[
            pl.BlockSpec((_BB, INST, _FB), lambda b, f: (b, 0, f)),
            pl.BlockSpec((INST, D_IN, _FB), lambda b, f: (0, 0, f)),
            pl.BlockSpec((INST, D_IN), lambda b, f: (0, 0)),
            pl.BlockSpec((1, 1), lambda b, f: (0, 0)),
        ],
        out_specs=[
            pl.BlockSpec((_BB, INST, _FB), lambda b, f: (b, 0, f)),
            pl.BlockSpec((_BB, INST, D_IN), lambda b, f: (b, 0, 0)),
        ],
        out_shape=[
            jax.ShapeDtypeStruct((BATCH, INST, d_level), jnp.float32),
            jax.ShapeDtypeStruct((BATCH, INST, D_IN), jnp.float32),
        ],
    )(acts, w_dec, b_dec, thr)


def kernel(x, W_enc, W_dec, b_enc, b_dec):
    xf = x.reshape(BATCH, INST, D_IN * x.shape[2])
    acts = _encoder(xf, W_enc, b_enc)

    thr = _sc_select(acts.reshape(ROWS, D_SAE))
    thr0 = thr[0, 0].reshape(1, 1)
    thr1 = thr[1, 0].reshape(1, 1)

    topk0, recon0 = _decoder(acts, W_dec, b_dec, thr0, D_LVL0)
    topk1, recon1 = _decoder(acts, W_dec, b_dec, thr1, D_SAE)

    return ((recon0, recon1), (topk0, topk1), topk1)


# speedup vs baseline: 16.3091x; 1.5691x over previous
import functools

import jax
import jax.numpy as jnp
from jax import lax
from jax.experimental import pallas as pl
from jax.experimental.pallas import tpu as pltpu
from jax.experimental.pallas import tpu_sc as plsc

BATCH = 512
INST = 2
D_IN = 256
D_SAE = 16384
D_LVL0 = 4096
K_LVL0 = BATCH * INST * 32
K_LVL1 = BATCH * INST * 64

ROWS = BATCH * INST
N_TILES = 16
ROWS_PER_TILE = ROWS // N_TILES
LANES = 16
HIST = 65536
CHUNK = HIST // N_TILES
CHUNK_V = CHUNK // LANES
_UNROLL = 4


_BB = 256
_FB = 2048


def _enc_body(x_ref, w_ref, b_ref, out_ref):
    for i in range(INST):
        acc = lax.dot_general(
            x_ref[:, i, :], w_ref[i], (((1,), (1,)), ((), ())),
            preferred_element_type=jnp.float32)
        acc = acc + b_ref[i][None, :]
        out_ref[:, i, :] = jnp.maximum(acc, 0.0)


def _encoder(xf, w_enc, b_enc):
    nb = BATCH // _BB
    nf = D_SAE // _FB
    return pl.pallas_call(
        _enc_body,
        grid=(nb, nf),
        in_specs=[
            pl.BlockSpec((_BB, INST, D_IN), lambda b, f: (b, 0, 0)),
            pl.BlockSpec((INST, _FB, D_IN), lambda b, f: (0, f, 0)),
            pl.BlockSpec((INST, _FB), lambda b, f: (0, f)),
        ],
        out_specs=pl.BlockSpec((_BB, INST, _FB), lambda b, f: (b, 0, f)),
        out_shape=jax.ShapeDtypeStruct((BATCH, INST, D_SAE), jnp.float32),
    )(xf, w_enc, b_enc)


def _sc_select(acts2):
    mesh = plsc.VectorSubcoreMesh(core_axis_name="c", subcore_axis_name="s")
    kern = pl.kernel(
        _sc_select_wrapped,
        out_type=(
            jax.ShapeDtypeStruct((INST, LANES), jnp.float32),
            jax.ShapeDtypeStruct((INST, N_TILES, N_TILES + 1, CHUNK), jnp.int32),
        ),
        mesh=mesh,
        compiler_params=pltpu.CompilerParams(needs_layout_passes=False),
        scratch_types=[
            pltpu.VMEM((D_SAE,), jnp.float32),
            pltpu.VMEM((N_TILES, CHUNK), jnp.int32),
            pltpu.VMEM((CHUNK,), jnp.int32),
            pltpu.VMEM((CHUNK,), jnp.int32),
            pltpu.VMEM((LANES,), jnp.int32),
            pltpu.VMEM((N_TILES, LANES), jnp.int32),
            pltpu.VMEM((LANES,), jnp.int32),
            pltpu.VMEM((LANES,), jnp.float32),
        ],
    )
    return kern(acts2)[0]


def _sc_select_wrapped(acts_hbm, out_hbm, hist_hbm, buf, hist, merged, tmp,
                       totv, totv2, resv, thrv):
    core = lax.axis_index("c")
    sub = lax.axis_index("s")

    n_vregs_row = jnp.where(core == 0, D_LVL0 // LANES, D_SAE // LANES)
    k_target = jnp.where(core == 0, jnp.int32(K_LVL0), jnp.int32(K_LVL1))
    zeros16 = jnp.zeros((LANES,), jnp.int32)
    iota = lax.iota(jnp.int32, LANES)

    def zero_ref(ref, n_v):
        def zbody(j, _):
            ref[pl.ds(j * LANES, LANES)] = zeros16
            return 0
        lax.fori_loop(0, n_v, zbody, 0)

    def extract_lane(vec, lane):
        sel = jnp.where(iota == lane, vec, jnp.zeros_like(vec))
        return lax.reduce_sum_p.bind(sel, axes=(0,))

    def splat(x):
        return jnp.full((LANES,), x, dtype=jnp.int32)

    def zero_hist():
        for r in range(N_TILES):
            def zbody(j, _):
                hist[r, pl.ds(j * LANES, LANES)] = zeros16
                return 0
            lax.fori_loop(0, CHUNK_V, zbody, 0)

    def stream_histogram(mode, b1):
        zero_hist()
        b1v = splat(b1)

        ones16 = jnp.ones((LANES,), jnp.int32)

        def row_body(rr, _):
            r = sub * ROWS_PER_TILE + rr
            pltpu.sync_copy(acts_hbm.at[r], buf)

            def vbody(j, _):
                for u in range(_UNROLL):
                    v = buf[pl.ds((j * _UNROLL + u) * LANES, LANES)]
                    pos = v > 0.0
                    bits = plsc.bitcast(v, jnp.int32)
                    hi = lax.shift_right_logical(bits, 16)
                    if mode == 0:
                        bkt = hi
                        m = pos
                    else:
                        bkt = lax.bitwise_and(bits, jnp.int32(0xFFFF))
                        m = jnp.logical_and(pos, hi == b1v)
                    row = lax.shift_right_logical(bkt, 12)
                    col = lax.bitwise_and(bkt, jnp.int32(CHUNK - 1))
                    plsc.addupdate_scatter(hist, [row, col], ones16, mask=m)
                return 0

            lax.fori_loop(0, n_vregs_row // _UNROLL, vbody, 0)
            return 0

        lax.fori_loop(0, ROWS_PER_TILE, row_body, 0)

    def merge_and_scan(k_want):
        pltpu.sync_copy(hist, hist_hbm.at[core, sub, pl.ds(0, N_TILES)])
        plsc.subcore_barrier()

        zero_ref(merged, CHUNK_V)

        def mbody(t, _):
            pltpu.sync_copy(hist_hbm.at[core, t, sub], tmp)

            def abody(j, _):
                s = pl.ds(j * LANES, LANES)
                merged[s] = merged[s] + tmp[s]
                return 0

            lax.fori_loop(0, CHUNK_V, abody, 0)
            return 0

        lax.fori_loop(0, N_TILES, mbody, 0)

        def tbody(j, acc):
            return acc + merged[pl.ds(j * LANES, LANES)]

        tot_vec = lax.fori_loop(0, CHUNK_V, tbody, zeros16)
        chunk_tot = lax.reduce_sum_p.bind(tot_vec, axes=(0,))
        totv[pl.ds(0, LANES)] = splat(chunk_tot)
        pltpu.sync_copy(totv, hist_hbm.at[core, sub, N_TILES, pl.ds(0, LANES)])
        plsc.subcore_barrier()

        for t in range(N_TILES):
            pltpu.sync_copy(hist_hbm.at[core, t, N_TILES, pl.ds(0, LANES)],
                            totv2.at[t])

        def sbody(i, carry):
            acc, cstar, above = carry
            t = N_TILES - 1 - i
            tvec = totv2[t]
            tt = extract_lane(tvec, jnp.int32(0))
            new_acc = acc + tt
            found = jnp.logical_and(cstar < 0, new_acc >= k_want)
            cstar = jnp.where(found, t, cstar)
            above = jnp.where(found, acc, above)
            return new_acc, cstar, above

        _, cstar, above = lax.fori_loop(
            0, N_TILES, sbody,
            (jnp.int32(0), jnp.int32(-1), jnp.int32(0)))

        def cbody(i, carry):
            cum, bsel, cgt = carry
            jv = CHUNK_V - 1 - i
            v = merged[pl.ds(jv * LANES, LANES)]
            rv = lax.rev(v, (0,))
            cs = plsc.cumsum(rv)
            cumincl = splat(cum) + cs
            crossed = cumincl >= splat(k_want)
            any_crossed = lax.reduce_max_p.bind(
                crossed.astype(jnp.int32), axes=(0,)) > 0
            lane = plsc.all_reduce_ffs(crossed)
            if lane.ndim != 0:
                lane = extract_lane(lane, jnp.int32(0))
            found = jnp.logical_and(bsel < 0, any_crossed)
            bucket = jv * LANES + (LANES - 1 - lane)
            cnt_at = extract_lane(rv, lane)
            cumincl_at = extract_lane(cumincl, lane)
            bsel = jnp.where(found, bucket, bsel)
            cgt = jnp.where(found, cumincl_at - cnt_at, cgt)
            cum = cum + lax.reduce_sum_p.bind(rv, axes=(0,))
            return cum, bsel, cgt

        _, bsel, cgt = lax.fori_loop(
            0, CHUNK_V, cbody,
            (above, jnp.int32(-1), jnp.int32(0)))

        @pl.when(sub == 0)
        def _():
            resv[pl.ds(0, LANES)] = zeros16
            pltpu.sync_copy(resv, hist_hbm.at[core, 0, N_TILES, pl.ds(LANES, LANES)])

        plsc.subcore_barrier()

        @pl.when(jnp.logical_and(cstar >= 0, sub == cstar))
        def _():
            bucket_g = cstar * CHUNK + bsel
            vec = jnp.where(iota == 0, splat(bucket_g),
                            jnp.where(iota == 1, splat(cgt),
                                      jnp.where(iota == 2, splat(1),
                                                zeros16)))
            resv[pl.ds(0, LANES)] = vec
            pltpu.sync_copy(resv, hist_hbm.at[core, 0, N_TILES, pl.ds(LANES, LANES)])

        plsc.subcore_barrier()
        pltpu.sync_copy(hist_hbm.at[core, 0, N_TILES, pl.ds(LANES, LANES)], resv)
        vec = resv[pl.ds(0, LANES)]
        bucket = extract_lane(vec, jnp.int32(0))
        count_gt = extract_lane(vec, jnp.int32(1))
        valid = extract_lane(vec, jnp.int32(2))
        return bucket, count_gt, valid, cstar

    stream_histogram(0, jnp.int32(0))
    b1, cgt1, valid1, _ = merge_and_scan(k_target)

    k_rem = k_target - cgt1
    stream_histogram(1, b1)
    b2, _, valid2, cstar2 = merge_and_scan(k_rem)

    thr_bits = lax.bitwise_or(lax.shift_left(b1, 16), b2)
    thr_vec = plsc.bitcast(splat(thr_bits), jnp.float32)
    ok = jnp.logical_and(valid1 > 0, valid2 > 0)
    thr_vec = jnp.where(ok, thr_vec, jnp.zeros((LANES,), jnp.float32))
    writer = jnp.where(cstar2 >= 0, cstar2, jnp.int32(0))

    @pl.when(sub == writer)
    def _():
        thrv[pl.ds(0, LANES)] = thr_vec
        pltpu.sync_copy(thrv, out_hbm.at[core])


def _dec_body(nf_total, acts_ref, w_ref, b_ref, thr_ref, topk_ref, recon_ref):
    nf = pl.program_id(1)
    t = thr_ref[0, 0]
    for i in range(INST):
        a = acts_ref[:, i, :]
        masked = jnp.where(a >= t, a, 0.0)
        topk_ref[:, i, :] = masked
        part = lax.dot_general(
            masked, w_ref[i], (((1,), (1,)), ((), ())),
            preferred_element_type=jnp.float32)

        @pl.when(nf == 0)
        def _():
            recon_ref[:, i, :] = part

        @pl.when(nf > 0)
        def _():
            recon_ref[:, i, :] += part

        @pl.when(nf == nf_total - 1)
        def _():
            recon_ref[:, i, :] = jnp.maximum(
                recon_ref[:, i, :] + b_ref[i][None, :], 0.0)


def _decoder(acts, w_dec, b_dec, thr, d_level):
    nb = BATCH // _BB
    nf = d_level // _FB
    return pl.pallas_call(
        functools.partial(_dec_body, nf),
        grid=(nb, nf),
        in_specs=[
            pl.BlockSpec((_BB, INST, _FB), lambda b, f: (b, 0, f)),
            pl.BlockSpec((INST, D_IN, _FB), lambda b, f: (0, 0, f)),
            pl.BlockSpec((INST, D_IN), lambda b, f: (0, 0)),
            pl.BlockSpec((1, 1), lambda b, f: (0, 0)),
        ],
        out_specs=[
            pl.BlockSpec((_BB, INST, _FB), lambda b, f: (b, 0, f)),
            pl.BlockSpec((_BB, INST, D_IN), lambda b, f: (b, 0, 0)),
        ],
        out_shape=[
            jax.ShapeDtypeStruct((BATCH, INST, d_level), jnp.float32),
            jax.ShapeDtypeStruct((BATCH, INST, D_IN), jnp.float32),
        ],
    )(acts, w_dec, b_dec, thr)


def kernel(x, W_enc, W_dec, b_enc, b_dec):
    xf = x.reshape(BATCH, INST, D_IN * x.shape[2])
    acts = _encoder(xf, W_enc, b_enc)

    thr = _sc_select(acts.reshape(ROWS, D_SAE))
    thr0 = thr[0, 0].reshape(1, 1)
    thr1 = thr[1, 0].reshape(1, 1)

    topk0, recon0 = _decoder(acts, W_dec, b_dec, thr0, D_LVL0)
    topk1, recon1 = _decoder(acts, W_dec, b_dec, thr1, D_SAE)

    return ((recon0, recon1), (topk0, topk1), topk1)


# speedup vs baseline: 16.4889x; 1.0110x over previous
import functools

import jax
import jax.numpy as jnp
from jax import lax
from jax.experimental import pallas as pl
from jax.experimental.pallas import tpu as pltpu
from jax.experimental.pallas import tpu_sc as plsc

BATCH = 512
INST = 2
D_IN = 256
D_SAE = 16384
D_LVL0 = 4096
K_LVL0 = BATCH * INST * 32
K_LVL1 = BATCH * INST * 64

ROWS = BATCH * INST
N_TILES = 16
ROWS_PER_TILE = ROWS // N_TILES
LANES = 16
HIST = 65536
CHUNK = HIST // N_TILES
CHUNK_V = CHUNK // LANES
_UNROLL = 8


_BB = 256
_FB = 2048


def _enc_body(x_ref, w_ref, b_ref, out_ref):
    for i in range(INST):
        acc = lax.dot_general(
            x_ref[:, i, :], w_ref[i], (((1,), (1,)), ((), ())),
            preferred_element_type=jnp.float32)
        acc = acc + b_ref[i][None, :]
        out_ref[:, i, :] = jnp.maximum(acc, 0.0)


def _encoder(xf, w_enc, b_enc):
    nb = BATCH // _BB
    nf = D_SAE // _FB
    return pl.pallas_call(
        _enc_body,
        grid=(nb, nf),
        in_specs=[
            pl.BlockSpec((_BB, INST, D_IN), lambda b, f: (b, 0, 0)),
            pl.BlockSpec((INST, _FB, D_IN), lambda b, f: (0, f, 0)),
            pl.BlockSpec((INST, _FB), lambda b, f: (0, f)),
        ],
        out_specs=pl.BlockSpec((_BB, INST, _FB), lambda b, f: (b, 0, f)),
        out_shape=jax.ShapeDtypeStruct((BATCH, INST, D_SAE), jnp.float32),
    )(xf, w_enc, b_enc)


def _sc_select(acts2):
    mesh = plsc.VectorSubcoreMesh(core_axis_name="c", subcore_axis_name="s")
    kern = pl.kernel(
        _sc_select_wrapped,
        out_type=(
            jax.ShapeDtypeStruct((INST, LANES), jnp.float32),
            jax.ShapeDtypeStruct((INST, N_TILES, N_TILES + 1, CHUNK), jnp.int32),
        ),
        mesh=mesh,
        compiler_params=pltpu.CompilerParams(needs_layout_passes=False),
        scratch_types=[
            pltpu.VMEM((D_SAE,), jnp.float32),
            pltpu.VMEM((N_TILES, CHUNK), jnp.int32),
            pltpu.VMEM((CHUNK,), jnp.int32),
            pltpu.VMEM((CHUNK,), jnp.int32),
            pltpu.VMEM((LANES,), jnp.int32),
            pltpu.VMEM((N_TILES, LANES), jnp.int32),
            pltpu.VMEM((LANES,), jnp.int32),
            pltpu.VMEM((LANES,), jnp.float32),
        ],
    )
    return kern(acts2)[0]


def _sc_select_wrapped(acts_hbm, out_hbm, hist_hbm, buf, hist, merged, tmp,
                       totv, totv2, resv, thrv):
    core = lax.axis_index("c")
    sub = lax.axis_index("s")

    n_vregs_row = jnp.where(core == 0, D_LVL0 // LANES, D_SAE // LANES)
    k_target = jnp.where(core == 0, jnp.int32(K_LVL0), jnp.int32(K_LVL1))
    zeros16 = jnp.zeros((LANES,), jnp.int32)
    iota = lax.iota(jnp.int32, LANES)

    def zero_ref(ref, n_v):
        def zbody(j, _):
            ref[pl.ds(j * LANES, LANES)] = zeros16
            return 0
        lax.fori_loop(0, n_v, zbody, 0)

    def extract_lane(vec, lane):
        sel = jnp.where(iota == lane, vec, jnp.zeros_like(vec))
        return lax.reduce_sum_p.bind(sel, axes=(0,))

    def splat(x):
        return jnp.full((LANES,), x, dtype=jnp.int32)

    def zero_hist():
        for r in range(N_TILES):
            def zbody(j, _):
                hist[r, pl.ds(j * LANES, LANES)] = zeros16
                return 0
            lax.fori_loop(0, CHUNK_V, zbody, 0)

    def stream_histogram(mode, b1):
        zero_hist()
        b1v = splat(b1)

        ones16 = jnp.ones((LANES,), jnp.int32)

        def row_body(rr, _):
            r = sub * ROWS_PER_TILE + rr
            pltpu.sync_copy(acts_hbm.at[r], buf)

            def vbody(j, _):
                for u in range(_UNROLL):
                    v = buf[pl.ds((j * _UNROLL + u) * LANES, LANES)]
                    pos = v > 0.0
                    bits = plsc.bitcast(v, jnp.int32)
                    hi = lax.shift_right_logical(bits, 16)
                    if mode == 0:
                        bkt = hi
                        m = pos
                    else:
                        bkt = lax.bitwise_and(bits, jnp.int32(0xFFFF))
                        m = jnp.logical_and(pos, hi == b1v)
                    row = lax.shift_right_logical(bkt, 12)
                    col = lax.bitwise_and(bkt, jnp.int32(CHUNK - 1))
                    plsc.addupdate_scatter(hist, [row, col], ones16, mask=m)
                return 0

            lax.fori_loop(0, n_vregs_row // _UNROLL, vbody, 0)
            return 0

        lax.fori_loop(0, ROWS_PER_TILE, row_body, 0)

    def merge_and_scan(k_want):
        pltpu.sync_copy(hist, hist_hbm.at[core, sub, pl.ds(0, N_TILES)])
        plsc.subcore_barrier()

        zero_ref(merged, CHUNK_V)

        def mbody(t, _):
            pltpu.sync_copy(hist_hbm.at[core, t, sub], tmp)

            def abody(j, _):
                s = pl.ds(j * LANES, LANES)
                merged[s] = merged[s] + tmp[s]
                return 0

            lax.fori_loop(0, CHUNK_V, abody, 0)
            return 0

        lax.fori_loop(0, N_TILES, mbody, 0)

        def tbody(j, acc):
            return acc + merged[pl.ds(j * LANES, LANES)]

        tot_vec = lax.fori_loop(0, CHUNK_V, tbody, zeros16)
        chunk_tot = lax.reduce_sum_p.bind(tot_vec, axes=(0,))
        totv[pl.ds(0, LANES)] = splat(chunk_tot)
        pltpu.sync_copy(totv, hist_hbm.at[core, sub, N_TILES, pl.ds(0, LANES)])
        plsc.subcore_barrier()

        for t in range(N_TILES):
            pltpu.sync_copy(hist_hbm.at[core, t, N_TILES, pl.ds(0, LANES)],
                            totv2.at[t])

        def sbody(i, carry):
            acc, cstar, above = carry
            t = N_TILES - 1 - i
            tvec = totv2[t]
            tt = extract_lane(tvec, jnp.int32(0))
            new_acc = acc + tt
            found = jnp.logical_and(cstar < 0, new_acc >= k_want)
            cstar = jnp.where(found, t, cstar)
            above = jnp.where(found, acc, above)
            return new_acc, cstar, above

        _, cstar, above = lax.fori_loop(
            0, N_TILES, sbody,
            (jnp.int32(0), jnp.int32(-1), jnp.int32(0)))

        def cbody(i, carry):
            cum, bsel, cgt = carry
            jv = CHUNK_V - 1 - i
            v = merged[pl.ds(jv * LANES, LANES)]
            rv = lax.rev(v, (0,))
            cs = plsc.cumsum(rv)
            cumincl = splat(cum) + cs
            crossed = cumincl >= splat(k_want)
            any_crossed = lax.reduce_max_p.bind(
                crossed.astype(jnp.int32), axes=(0,)) > 0
            lane = plsc.all_reduce_ffs(crossed)
            if lane.ndim != 0:
                lane = extract_lane(lane, jnp.int32(0))
            found = jnp.logical_and(bsel < 0, any_crossed)
            bucket = jv * LANES + (LANES - 1 - lane)
            cnt_at = extract_lane(rv, lane)
            cumincl_at = extract_lane(cumincl, lane)
            bsel = jnp.where(found, bucket, bsel)
            cgt = jnp.where(found, cumincl_at - cnt_at, cgt)
            cum = cum + lax.reduce_sum_p.bind(rv, axes=(0,))
            return cum, bsel, cgt

        _, bsel, cgt = lax.fori_loop(
            0, CHUNK_V, cbody,
            (above, jnp.int32(-1), jnp.int32(0)))

        @pl.when(sub == 0)
        def _():
            resv[pl.ds(0, LANES)] = zeros16
            pltpu.sync_copy(resv, hist_hbm.at[core, 0, N_TILES, pl.ds(LANES, LANES)])

        plsc.subcore_barrier()

        @pl.when(jnp.logical_and(cstar >= 0, sub == cstar))
        def _():
            bucket_g = cstar * CHUNK + bsel
            vec = jnp.where(iota == 0, splat(bucket_g),
                            jnp.where(iota == 1, splat(cgt),
                                      jnp.where(iota == 2, splat(1),
                                                zeros16)))
            resv[pl.ds(0, LANES)] = vec
            pltpu.sync_copy(resv, hist_hbm.at[core, 0, N_TILES, pl.ds(LANES, LANES)])

        plsc.subcore_barrier()
        pltpu.sync_copy(hist_hbm.at[core, 0, N_TILES, pl.ds(LANES, LANES)], resv)
        vec = resv[pl.ds(0, LANES)]
        bucket = extract_lane(vec, jnp.int32(0))
        count_gt = extract_lane(vec, jnp.int32(1))
        valid = extract_lane(vec, jnp.int32(2))
        return bucket, count_gt, valid, cstar

    stream_histogram(0, jnp.int32(0))
    b1, cgt1, valid1, _ = merge_and_scan(k_target)

    k_rem = k_target - cgt1
    stream_histogram(1, b1)
    b2, _, valid2, cstar2 = merge_and_scan(k_rem)

    thr_bits = lax.bitwise_or(lax.shift_left(b1, 16), b2)
    thr_vec = plsc.bitcast(splat(thr_bits), jnp.float32)
    ok = jnp.logical_and(valid1 > 0, valid2 > 0)
    thr_vec = jnp.where(ok, thr_vec, jnp.zeros((LANES,), jnp.float32))
    writer = jnp.where(cstar2 >= 0, cstar2, jnp.int32(0))

    @pl.when(sub == writer)
    def _():
        thrv[pl.ds(0, LANES)] = thr_vec
        pltpu.sync_copy(thrv, out_hbm.at[core])


def _dec_body(nf_total, acts_ref, w_ref, b_ref, thr_ref, topk_ref, recon_ref):
    nf = pl.program_id(1)
    t = thr_ref[0, 0]
    for i in range(INST):
        a = acts_ref[:, i, :]
        masked = jnp.where(a >= t, a, 0.0)
        topk_ref[:, i, :] = masked
        part = lax.dot_general(
            masked, w_ref[i], (((1,), (1,)), ((), ())),
            preferred_element_type=jnp.float32)

        @pl.when(nf == 0)
        def _():
            recon_ref[:, i, :] = part

        @pl.when(nf > 0)
        def _():
            recon_ref[:, i, :] += part

        @pl.when(nf == nf_total - 1)
        def _():
            recon_ref[:, i, :] = jnp.maximum(
                recon_ref[:, i, :] + b_ref[i][None, :], 0.0)


def _decoder(acts, w_dec, b_dec, thr, d_level):
    nb = BATCH // _BB
    nf = d_level // _FB
    return pl.pallas_call(
        functools.partial(_dec_body, nf),
        grid=(nb, nf),
        in_specs=[
            pl.BlockSpec((_BB, INST, _FB), lambda b, f: (b, 0, f)),
            pl.BlockSpec((INST, D_IN, _FB), lambda b, f: (0, 0, f)),
            pl.BlockSpec((INST, D_IN), lambda b, f: (0, 0)),
            pl.BlockSpec((1, 1), lambda b, f: (0, 0)),
        ],
        out_specs=[
            pl.BlockSpec((_BB, INST, _FB), lambda b, f: (b, 0, f)),
            pl.BlockSpec((_BB, INST, D_IN), lambda b, f: (b, 0, 0)),
        ],
        out_shape=[
            jax.ShapeDtypeStruct((BATCH, INST, d_level), jnp.float32),
            jax.ShapeDtypeStruct((BATCH, INST, D_IN), jnp.float32),
        ],
    )(acts, w_dec, b_dec, thr)


def kernel(x, W_enc, W_dec, b_enc, b_dec):
    xf = x.reshape(BATCH, INST, D_IN * x.shape[2])
    acts = _encoder(xf, W_enc, b_enc)

    thr = _sc_select(acts.reshape(ROWS, D_SAE))
    thr0 = thr[0, 0].reshape(1, 1)
    thr1 = thr[1, 0].reshape(1, 1)

    topk0, recon0 = _decoder(acts, W_dec, b_dec, thr0, D_LVL0)
    topk1, recon1 = _decoder(acts, W_dec, b_dec, thr1, D_SAE)

    return ((recon0, recon1), (topk0, topk1), topk1)
